# baseline (speedup 1.0000x reference)
import functools

import jax
import jax.numpy as jnp
from jax import lax
from jax.experimental import pallas as pl
from jax.experimental.pallas import tpu as pltpu
from jax.experimental.pallas import tpu_sc as plsc

_mesh = plsc.ScalarSubcoreMesh(axis_name="c", num_cores=2)


@functools.partial(
    pl.kernel,
    mesh=_mesh,
    out_type=jax.ShapeDtypeStruct((1, 2, 64, 128), jnp.float32),
    scratch_types=[
        pltpu.SMEM((16,), jnp.int32),
        pltpu.SemaphoreType.DMA,
        pltpu.SemaphoreType.DMA,
    ],
)
def _extract(table_hbm, step_hbm, out_hbm, step_s, sem0, sem1):
    c = lax.axis_index("c")
    pltpu.sync_copy(step_hbm, step_s)
    s = step_s[0]
    cp0 = pltpu.make_async_copy(
        table_hbm.at[pl.ds(s, 1), c, pl.ds(0, 32)],
        out_hbm.at[pl.ds(0, 1), c, pl.ds(0, 32)], sem0)
    cp1 = pltpu.make_async_copy(
        table_hbm.at[pl.ds(s, 1), c, pl.ds(32, 32)],
        out_hbm.at[pl.ds(0, 1), c, pl.ds(32, 32)], sem1)
    cp0.start()
    cp1.start()
    cp0.wait()
    cp1.wait()


def kernel(table, step):
    step_vec = jnp.full((16,), step, dtype=jnp.int32)
    tview = jnp.swapaxes(table, 2, 3)
    out = _extract(tview, step_vec)
    return jnp.swapaxes(out, 2, 3)
